# baseline (speedup 1.0000x reference)
import functools

import jax
import jax.numpy as jnp
from jax import lax
from jax.experimental import pallas as pl
from jax.experimental.pallas import tpu as pltpu
from jax.experimental.pallas import tpu_sc as plsc

_F = 26
_V = 100000
_D = 32
_B = 4096


@functools.cache
def _build():
    info = plsc.get_sparse_core_info()
    nw = info.num_cores * info.num_subcores
    bw = _B // nw
    assert bw * nw == _B

    mesh = plsc.VectorSubcoreMesh(core_axis_name="c", subcore_axis_name="s")

    @functools.partial(
        pl.kernel,
        mesh=mesh,
        compiler_params=pltpu.CompilerParams(use_tc_tiling_on_sc=False),
        out_type=jax.ShapeDtypeStruct((_B, _F, _D), jnp.float32),
        scratch_types=[
            pltpu.VMEM((_F, bw), jnp.int32),
            pltpu.VMEM((_F, bw, _D), jnp.float32),
            pltpu.SemaphoreType.DMA,
            pltpu.SemaphoreType.DMA,
        ],
    )
    def sc_gather(xt_hbm, tab_hbm, out_hbm, idx_v, rows_v, gsem, wsem):
        wid = lax.axis_index("s") * info.num_cores + lax.axis_index("c")
        base = wid * bw
        pltpu.sync_copy(xt_hbm.at[:, pl.ds(base, bw)], idx_v)

        def body(f, carry):
            pltpu.async_copy(
                tab_hbm.at[f].at[idx_v.at[f]], rows_v.at[f], gsem
            ).wait()
            pltpu.async_copy(
                rows_v.at[f], out_hbm.at[pl.ds(base, bw), f], wsem
            )
            return carry

        lax.fori_loop(0, _F, body, 0)
        pltpu.make_async_copy(
            out_hbm.at[pl.ds(base, bw)], rows_v, wsem
        ).wait()

    return sc_gather


def kernel(x, tables):
    out = _build()(x.T, tables)
    return out.reshape(_B, _F * _D)
